# baseline (speedup 1.0000x reference)
import functools
import math

import jax
import jax.numpy as jnp
from jax import lax
from jax.experimental import pallas as pl
from jax.experimental.pallas import tpu as pltpu
from jax.experimental.pallas import tpu_sc as plsc

N = 10000
E = 320000
D = 128
H = 128
C = 10
G = 128
EPS = 1e-5
_RS = 1.0 / math.sqrt(1.0 + EPS)

NC = 2
NS = 16
HH = H // NC
NP = 10240
ZRA = 632
ZRB = N - (NS - 1) * ZRA
CK = 256
NCH = E // NS // CK
E2 = NS * NCH * CK
RCK = 128
NREM = (E - E2) // RCK


def _mesh():
    return plsc.VectorSubcoreMesh(
        core_axis_name="c", subcore_axis_name="s", num_cores=NC, num_subcores=NS
    )


@functools.partial(
    pl.kernel,
    out_type=jax.ShapeDtypeStruct((NC * NP,), jnp.float32),
    mesh=_mesh(),
    compiler_params=pltpu.CompilerParams(use_tc_tiling_on_sc=False),
    scratch_types=[
        pltpu.VMEM((NCH, CK), jnp.int32),
        pltpu.VMEM((CK,), jnp.float32),
        pltpu.VMEM_SHARED((NP,), jnp.float32),
    ],
)
def _deg_sc(r2_hbm, ones_hbm, zeros_hbm, out_hbm, idx_v, ones_v, acc_sh):
    c = lax.axis_index("c")
    s = lax.axis_index("s")
    pltpu.sync_copy(ones_hbm, ones_v)

    @pl.when(s == 0)
    def _():
        pltpu.sync_copy(zeros_hbm, acc_sh)

    plsc.subcore_barrier()
    pltpu.sync_copy(r2_hbm.at[s], idx_v)

    def body(j, carry):
        pltpu.sync_copy(ones_v, acc_sh.at[idx_v.at[j]], add=True)
        return carry

    lax.fori_loop(0, NCH, body, 0)
    plsc.subcore_barrier()

    @pl.when(s == 0)
    def _():
        pltpu.sync_copy(acc_sh, out_hbm.at[pl.ds(c * NP, NP)])


@functools.partial(
    pl.kernel,
    out_type=jax.ShapeDtypeStruct((NC, N, HH), jnp.float32),
    mesh=_mesh(),
    compiler_params=pltpu.CompilerParams(use_tc_tiling_on_sc=False),
    scratch_types=[
        pltpu.VMEM((NCH, CK), jnp.int32),
        pltpu.VMEM((NCH, CK), jnp.int32),
        pltpu.VMEM((NREM, RCK), jnp.int32),
        pltpu.VMEM((NREM, RCK), jnp.int32),
        pltpu.VMEM((CK, HH), jnp.float32),
        pltpu.VMEM((CK, HH), jnp.float32),
        pltpu.VMEM((RCK, HH), jnp.float32),
        pltpu.VMEM_SHARED((N, HH), jnp.float32),
        pltpu.SemaphoreType.DMA,
        pltpu.SemaphoreType.DMA,
        pltpu.SemaphoreType.DMA,
        pltpu.SemaphoreType.DMA,
    ],
)
def _prop_sc(q_hbm, r2_hbm, c2_hbm, rr_hbm, cr_hbm, zrows_hbm, out_hbm,
             ir_v, ic_v, irr_v, icr_v, rows0, rows1, rowsr, acc_sh,
             gsem0, gsem1, ssem0, ssem1):
    c = lax.axis_index("c")
    s = lax.axis_index("s")

    @pl.when(s < NS - 1)
    def _():
        pltpu.sync_copy(zrows_hbm, acc_sh.at[pl.ds(s * ZRA, ZRA)])

    @pl.when(s == NS - 1)
    def _():
        pltpu.sync_copy(
            zrows_hbm.at[pl.ds(0, ZRB)], acc_sh.at[pl.ds((NS - 1) * ZRA, ZRB)]
        )

    plsc.subcore_barrier()
    qh = q_hbm.at[c]
    pltpu.sync_copy(r2_hbm.at[s], ir_v)
    pltpu.sync_copy(c2_hbm.at[s], ic_v)
    pltpu.async_copy(qh.at[ir_v.at[0]], rows0, gsem0)

    def body(t, carry):
        j0 = 2 * t
        pltpu.make_async_copy(qh.at[ir_v.at[j0]], rows0, gsem0).wait()
        pltpu.async_copy(rows0, acc_sh.at[ic_v.at[j0]], ssem0, add=True)

        @pl.when(t > 0)
        def _():
            pltpu.make_async_copy(rows1, acc_sh.at[ic_v.at[0]], ssem1).wait()

        pltpu.async_copy(qh.at[ir_v.at[j0 + 1]], rows1, gsem1)

        pltpu.make_async_copy(qh.at[ir_v.at[j0 + 1]], rows1, gsem1).wait()
        pltpu.async_copy(rows1, acc_sh.at[ic_v.at[j0 + 1]], ssem1, add=True)
        pltpu.make_async_copy(rows0, acc_sh.at[ic_v.at[0]], ssem0).wait()

        @pl.when(j0 + 2 < NCH)
        def _():
            pltpu.async_copy(qh.at[ir_v.at[j0 + 2]], rows0, gsem0)

        return carry

    lax.fori_loop(0, NCH // 2, body, 0)
    pltpu.make_async_copy(rows1, acc_sh.at[ic_v.at[0]], ssem1).wait()

    @pl.when(s == 0)
    def _():
        pltpu.sync_copy(rr_hbm, irr_v)
        pltpu.sync_copy(cr_hbm, icr_v)

        def rbody(t, carry):
            pltpu.async_copy(qh.at[irr_v.at[t]], rowsr, gsem0).wait()
            pltpu.sync_copy(rowsr, acc_sh.at[icr_v.at[t]], add=True)
            return carry

        lax.fori_loop(0, NREM, rbody, 0)

    plsc.subcore_barrier()

    @pl.when(s < NS - 1)
    def _():
        pltpu.sync_copy(
            acc_sh.at[pl.ds(s * ZRA, ZRA)], out_hbm.at[c].at[pl.ds(s * ZRA, ZRA)]
        )

    @pl.when(s == NS - 1)
    def _():
        pltpu.sync_copy(
            acc_sh.at[pl.ds((NS - 1) * ZRA, ZRB)],
            out_hbm.at[c].at[pl.ds((NS - 1) * ZRA, ZRB)],
        )


NB = 10
BN = N // NB

_bspec_h = pl.BlockSpec((BN, H), lambda i: (i, 0))
_bspec_d = pl.BlockSpec((BN, 1), lambda i: (i, 0))
_bspec_w = pl.BlockSpec((H, H), lambda i: (0, 0))
_bspec_v = pl.BlockSpec((1, H), lambda i: (0, 0))
_bspec_q = pl.BlockSpec((NC, BN, HH), lambda i: (0, i, 0))


EREM = E - E2


def _tcq_body(x_ref, deg_ref, rr_ref, gf_ref, bf_ref, wf_ref, g_ref, b_ref,
              w_ref, q_ref, degc_ref):
    i = pl.program_id(0)
    ids = lax.broadcasted_iota(jnp.int32, (EREM, BN), 1) + i * BN
    oh = jnp.where(rr_ref[:] == ids, 1.0, 0.0)
    extra = lax.dot_general(oh, jnp.ones((EREM, 1), jnp.float32),
                            (((0,), (0,)), ((), ())),
                            preferred_element_type=jnp.float32)
    degc = deg_ref[:] + extra
    degc_ref[:] = degc
    dis = lax.rsqrt(degc + 1.0)
    t0 = x_ref[:] * (_RS * gf_ref[:]) + bf_ref[:]
    h0 = jnp.maximum(jnp.dot(t0, wf_ref[:], preferred_element_type=jnp.float32), 0.0)
    t = h0 * (_RS * g_ref[:]) + b_ref[:]
    q = dis * jnp.dot(t, w_ref[:], preferred_element_type=jnp.float32)
    q_ref[0] = q[:, :HH]
    q_ref[1] = q[:, HH:]


_tcq = pl.pallas_call(
    _tcq_body,
    grid=(NB,),
    in_specs=[_bspec_h, _bspec_d, pl.BlockSpec((EREM, 1), lambda i: (0, 0)),
              _bspec_v, _bspec_v, _bspec_w, _bspec_v, _bspec_v, _bspec_w],
    out_specs=(_bspec_q, _bspec_d),
    out_shape=(jax.ShapeDtypeStruct((NC, N, HH), jnp.float32),
               jax.ShapeDtypeStruct((N, 1), jnp.float32)),
)


def _tcmid_body(accp_ref, q_ref, deg_ref, bp_ref, g_ref, b_ref, w_ref, o_ref):
    dis = lax.rsqrt(deg_ref[:] + 1.0)
    m = jnp.concatenate([accp_ref[0] + q_ref[0], accp_ref[1] + q_ref[1]], axis=1)
    h = jnp.maximum(dis * m + bp_ref[:], 0.0)
    t = h * (_RS * g_ref[:]) + b_ref[:]
    r = dis * jnp.dot(t, w_ref[:], preferred_element_type=jnp.float32)
    o_ref[0] = r[:, :HH]
    o_ref[1] = r[:, HH:]


_tcmid = pl.pallas_call(
    _tcmid_body,
    grid=(NB,),
    in_specs=[_bspec_q, _bspec_q, _bspec_d, _bspec_v, _bspec_v, _bspec_v,
              _bspec_w],
    out_specs=_bspec_q,
    out_shape=jax.ShapeDtypeStruct((NC, N, HH), jnp.float32),
)


def _tcfin_body(accp_ref, q_ref, deg_ref, bp_ref, batch_ref, gfc_ref, bfc_ref,
                wl_ref, bl_ref, gh_ref, bh_ref, wc_ref, bc_ref, o_ref):
    dis = lax.rsqrt(deg_ref[:] + 1.0)
    m = jnp.concatenate([accp_ref[0] + q_ref[0], accp_ref[1] + q_ref[1]], axis=1)
    h = jnp.maximum(dis * m + bp_ref[:], 0.0)
    ids = lax.broadcasted_iota(jnp.int32, (N, G), 1)
    onehot = jnp.where(batch_ref[:] == ids, 1.0, 0.0)
    dn = (((0,), (0,)), ((), ()))
    sums = lax.dot_general(onehot, h, dn, preferred_element_type=jnp.float32)
    cnt = lax.dot_general(
        onehot, jnp.ones((N, 1), jnp.float32), dn, preferred_element_type=jnp.float32
    )
    pooled = sums / jnp.maximum(cnt, 1.0)
    a = pooled * (_RS * gfc_ref[:]) + bfc_ref[:]
    a = jnp.maximum(
        jnp.dot(a, wl_ref[:], preferred_element_type=jnp.float32) + bl_ref[:], 0.0
    )
    a = a * (_RS * gh_ref[:]) + bh_ref[:]
    lg = jnp.dot(a, wc_ref[:], preferred_element_type=jnp.float32) + bc_ref[:]
    m = jnp.max(lg, axis=1, keepdims=True)
    o_ref[:] = (lg - m) - jnp.log(jnp.sum(jnp.exp(lg - m), axis=1, keepdims=True))


_tcfin = pl.pallas_call(_tcfin_body, out_shape=jax.ShapeDtypeStruct((G, C), jnp.float32))


def kernel(x, edge_index, batch, bnf_g, bnf_b, Wf, bn1_g, bn1_b, W1, b1,
           bn2_g, bn2_b, W2, b2, bn3_g, bn3_b, W3, b3, bnfc_g, bnfc_b, Wl, bl,
           bnh_g, bnh_b, Wc, bc):
    row = edge_index[0]
    col = edge_index[1]
    row2d = row[:E2].reshape(NS, NCH, CK)
    col2d = col[:E2].reshape(NS, NCH, CK)
    rowrem = row[E2:].reshape(NREM, RCK)
    colrem = col[E2:].reshape(NREM, RCK)
    r2 = lambda v: v.reshape(1, -1)
    ones = jnp.ones((CK,), jnp.float32)
    z_n = jnp.zeros((NP,), jnp.float32)
    z_rows = jnp.zeros((ZRA, HH), jnp.float32)

    degp = _deg_sc(row2d, ones, z_n).reshape(NC, NP)[0, :N].reshape(N, 1)
    q1, degc = _tcq(x, degp, row[E2:].reshape(E - E2, 1), r2(bnf_g), r2(bnf_b),
                    Wf, r2(bn1_g), r2(bn1_b), W1)
    acc1 = _prop_sc(q1, row2d, col2d, rowrem, colrem, z_rows)
    q2 = _tcmid(acc1, q1, degc, r2(b1), r2(bn2_g), r2(bn2_b), W2)
    acc2 = _prop_sc(q2, row2d, col2d, rowrem, colrem, z_rows)
    q3 = _tcmid(acc2, q2, degc, r2(b2), r2(bn3_g), r2(bn3_b), W3)
    acc3 = _prop_sc(q3, row2d, col2d, rowrem, colrem, z_rows)
    return _tcfin(acc3, q3, degc, r2(b3), batch.reshape(N, 1),
                  r2(bnfc_g), r2(bnfc_b), Wl, r2(bl), r2(bnh_g), r2(bnh_b),
                  Wc, r2(bc))


# speedup vs baseline: 1.3463x; 1.3463x over previous
import functools
import math

import jax
import jax.numpy as jnp
from jax import lax
from jax.experimental import pallas as pl
from jax.experimental.pallas import tpu as pltpu
from jax.experimental.pallas import tpu_sc as plsc

N = 10000
E = 320000
D = 128
H = 128
C = 10
G = 128
EPS = 1e-5
_RS = 1.0 / math.sqrt(1.0 + EPS)

NC = 2
NS = 16
HH = H // NC
NP = 10240
ZRA = 632
ZRB = N - (NS - 1) * ZRA
CK = 256
NCH = E // NS // CK
E2 = NS * NCH * CK
NREM = (E - E2) // CK


def _mesh():
    return plsc.VectorSubcoreMesh(
        core_axis_name="c", subcore_axis_name="s", num_cores=NC, num_subcores=NS
    )


@functools.partial(
    pl.kernel,
    out_type=jax.ShapeDtypeStruct((NC * NP,), jnp.float32),
    mesh=_mesh(),
    compiler_params=pltpu.CompilerParams(use_tc_tiling_on_sc=False),
    scratch_types=[
        pltpu.VMEM((NCH, CK), jnp.int32),
        pltpu.VMEM((NREM, CK), jnp.int32),
        pltpu.VMEM((CK,), jnp.float32),
        pltpu.VMEM_SHARED((NP,), jnp.float32),
    ],
)
def _deg_sc(r2_hbm, rr_hbm, ones_hbm, zeros_hbm, out_hbm, idx_v, idxr_v, ones_v, acc_sh):
    c = lax.axis_index("c")
    s = lax.axis_index("s")
    pltpu.sync_copy(ones_hbm, ones_v)

    @pl.when(s == 0)
    def _():
        pltpu.sync_copy(zeros_hbm, acc_sh)

    plsc.subcore_barrier()
    pltpu.sync_copy(r2_hbm.at[s], idx_v)

    def body(j, carry):
        pltpu.sync_copy(ones_v, acc_sh.at[idx_v.at[j]], add=True)
        return carry

    lax.fori_loop(0, NCH, body, 0)

    @pl.when(s == 0)
    def _():
        pltpu.sync_copy(rr_hbm, idxr_v)

        def rbody(j, carry):
            pltpu.sync_copy(ones_v, acc_sh.at[idxr_v.at[j]], add=True)
            return carry

        lax.fori_loop(0, NREM, rbody, 0)

    plsc.subcore_barrier()

    @pl.when(s == 0)
    def _():
        pltpu.sync_copy(acc_sh, out_hbm.at[pl.ds(c * NP, NP)])


@functools.partial(
    pl.kernel,
    out_type=jax.ShapeDtypeStruct((NC, N, HH), jnp.bfloat16),
    mesh=_mesh(),
    compiler_params=pltpu.CompilerParams(use_tc_tiling_on_sc=False),
    scratch_types=[
        pltpu.VMEM((NCH, CK), jnp.int32),
        pltpu.VMEM((NCH, CK), jnp.int32),
        pltpu.VMEM((NREM, CK), jnp.int32),
        pltpu.VMEM((NREM, CK), jnp.int32),
        pltpu.VMEM((CK, HH), jnp.bfloat16),
        pltpu.VMEM((CK, HH), jnp.bfloat16),
        pltpu.VMEM_SHARED((N, HH), jnp.bfloat16),
        pltpu.SemaphoreType.DMA,
        pltpu.SemaphoreType.DMA,
    ],
)
def _prop_sc(q_hbm, r2_hbm, c2_hbm, rr_hbm, cr_hbm, zrows_hbm, out_hbm,
             ir_v, ic_v, irr_v, icr_v, rows0, rows1, acc_sh, sem0, sem1):
    c = lax.axis_index("c")
    s = lax.axis_index("s")

    @pl.when(s < NS - 1)
    def _():
        pltpu.sync_copy(zrows_hbm, acc_sh.at[pl.ds(s * ZRA, ZRA)])

    @pl.when(s == NS - 1)
    def _():
        pltpu.sync_copy(
            zrows_hbm.at[pl.ds(0, ZRB)], acc_sh.at[pl.ds((NS - 1) * ZRA, ZRB)]
        )

    plsc.subcore_barrier()
    qh = q_hbm.at[c]
    pltpu.sync_copy(r2_hbm.at[s], ir_v)
    pltpu.sync_copy(c2_hbm.at[s], ic_v)
    pltpu.async_copy(qh.at[ir_v.at[0]], rows0, sem0)

    def body(t, carry):
        j0 = 2 * t
        pltpu.async_copy(qh.at[ir_v.at[j0 + 1]], rows1, sem1)
        pltpu.make_async_copy(qh.at[ir_v.at[j0]], rows0, sem0).wait()
        pltpu.sync_copy(rows0, acc_sh.at[ic_v.at[j0]], add=True)

        @pl.when(j0 + 2 < NCH)
        def _():
            pltpu.async_copy(qh.at[ir_v.at[j0 + 2]], rows0, sem0)

        pltpu.make_async_copy(qh.at[ir_v.at[j0 + 1]], rows1, sem1).wait()
        pltpu.sync_copy(rows1, acc_sh.at[ic_v.at[j0 + 1]], add=True)
        return carry

    lax.fori_loop(0, NCH // 2, body, 0)

    if NCH % 2 == 1:
        pltpu.make_async_copy(qh.at[ir_v.at[NCH - 1]], rows0, sem0).wait()
        pltpu.sync_copy(rows0, acc_sh.at[ic_v.at[NCH - 1]], add=True)

    @pl.when(s == 0)
    def _():
        pltpu.sync_copy(rr_hbm, irr_v)
        pltpu.sync_copy(cr_hbm, icr_v)

        def rbody(t, carry):
            pltpu.async_copy(qh.at[irr_v.at[t]], rows0, sem0).wait()
            pltpu.sync_copy(rows0, acc_sh.at[icr_v.at[t]], add=True)
            return carry

        lax.fori_loop(0, NREM, rbody, 0)

    plsc.subcore_barrier()

    @pl.when(s < NS - 1)
    def _():
        pltpu.sync_copy(
            acc_sh.at[pl.ds(s * ZRA, ZRA)], out_hbm.at[c].at[pl.ds(s * ZRA, ZRA)]
        )

    @pl.when(s == NS - 1)
    def _():
        pltpu.sync_copy(
            acc_sh.at[pl.ds((NS - 1) * ZRA, ZRB)],
            out_hbm.at[c].at[pl.ds((NS - 1) * ZRA, ZRB)],
        )


NB = 10
BN = N // NB

_bspec_h = pl.BlockSpec((BN, H), lambda i: (i, 0))
_bspec_d = pl.BlockSpec((BN, 1), lambda i: (i, 0))
_bspec_w = pl.BlockSpec((H, H), lambda i: (0, 0))
_bspec_v = pl.BlockSpec((1, H), lambda i: (0, 0))
_bspec_q = pl.BlockSpec((NC, BN, HH), lambda i: (0, i, 0))


def _tcq_body(x_ref, deg_ref, gf_ref, bf_ref, wf_ref, g_ref, b_ref, w_ref, q_ref, qb_ref):
    dis = lax.rsqrt(deg_ref[:] + 1.0)
    t0 = x_ref[:] * (_RS * gf_ref[:]) + bf_ref[:]
    h0 = jnp.maximum(jnp.dot(t0, wf_ref[:], preferred_element_type=jnp.float32), 0.0)
    t = h0 * (_RS * g_ref[:]) + b_ref[:]
    q = dis * jnp.dot(t, w_ref[:], preferred_element_type=jnp.float32)
    q_ref[0] = q[:, :HH]
    q_ref[1] = q[:, HH:]
    qb_ref[0] = q[:, :HH].astype(jnp.bfloat16)
    qb_ref[1] = q[:, HH:].astype(jnp.bfloat16)


_tcq = pl.pallas_call(
    _tcq_body,
    grid=(NB,),
    in_specs=[_bspec_h, _bspec_d, _bspec_v, _bspec_v, _bspec_w, _bspec_v,
              _bspec_v, _bspec_w],
    out_specs=(_bspec_q, _bspec_q),
    out_shape=(jax.ShapeDtypeStruct((NC, N, HH), jnp.float32),
               jax.ShapeDtypeStruct((NC, N, HH), jnp.bfloat16)),
)


def _tcmid_body(accp_ref, q_ref, deg_ref, bp_ref, g_ref, b_ref, w_ref,
                o_ref, ob_ref):
    dis = lax.rsqrt(deg_ref[:] + 1.0)
    m = jnp.concatenate(
        [accp_ref[0].astype(jnp.float32) + q_ref[0],
         accp_ref[1].astype(jnp.float32) + q_ref[1]], axis=1)
    h = jnp.maximum(dis * m + bp_ref[:], 0.0)
    t = h * (_RS * g_ref[:]) + b_ref[:]
    r = dis * jnp.dot(t, w_ref[:], preferred_element_type=jnp.float32)
    o_ref[0] = r[:, :HH]
    o_ref[1] = r[:, HH:]
    ob_ref[0] = r[:, :HH].astype(jnp.bfloat16)
    ob_ref[1] = r[:, HH:].astype(jnp.bfloat16)


_tcmid = pl.pallas_call(
    _tcmid_body,
    grid=(NB,),
    in_specs=[_bspec_q, _bspec_q, _bspec_d, _bspec_v, _bspec_v, _bspec_v,
              _bspec_w],
    out_specs=(_bspec_q, _bspec_q),
    out_shape=(jax.ShapeDtypeStruct((NC, N, HH), jnp.float32),
               jax.ShapeDtypeStruct((NC, N, HH), jnp.bfloat16)),
)


def _tcfin_body(accp_ref, q_ref, deg_ref, bp_ref, batch_ref, gfc_ref, bfc_ref,
                wl_ref, bl_ref, gh_ref, bh_ref, wc_ref, bc_ref, o_ref):
    dis = lax.rsqrt(deg_ref[:] + 1.0)
    m = jnp.concatenate(
        [accp_ref[0].astype(jnp.float32) + q_ref[0],
         accp_ref[1].astype(jnp.float32) + q_ref[1]], axis=1)
    h = jnp.maximum(dis * m + bp_ref[:], 0.0)
    ids = lax.broadcasted_iota(jnp.int32, (N, G), 1)
    onehot = jnp.where(batch_ref[:] == ids, 1.0, 0.0)
    dn = (((0,), (0,)), ((), ()))
    sums = lax.dot_general(onehot, h, dn, preferred_element_type=jnp.float32)
    cnt = lax.dot_general(
        onehot, jnp.ones((N, 1), jnp.float32), dn, preferred_element_type=jnp.float32
    )
    pooled = sums / jnp.maximum(cnt, 1.0)
    a = pooled * (_RS * gfc_ref[:]) + bfc_ref[:]
    a = jnp.maximum(
        jnp.dot(a, wl_ref[:], preferred_element_type=jnp.float32) + bl_ref[:], 0.0
    )
    a = a * (_RS * gh_ref[:]) + bh_ref[:]
    lg = jnp.dot(a, wc_ref[:], preferred_element_type=jnp.float32) + bc_ref[:]
    m = jnp.max(lg, axis=1, keepdims=True)
    o_ref[:] = (lg - m) - jnp.log(jnp.sum(jnp.exp(lg - m), axis=1, keepdims=True))


_tcfin = pl.pallas_call(_tcfin_body, out_shape=jax.ShapeDtypeStruct((G, C), jnp.float32))


def kernel(x, edge_index, batch, bnf_g, bnf_b, Wf, bn1_g, bn1_b, W1, b1,
           bn2_g, bn2_b, W2, b2, bn3_g, bn3_b, W3, b3, bnfc_g, bnfc_b, Wl, bl,
           bnh_g, bnh_b, Wc, bc):
    row = edge_index[0]
    col = edge_index[1]
    row2d = row[:E2].reshape(NS, NCH, CK)
    col2d = col[:E2].reshape(NS, NCH, CK)
    rowrem = row[E2:].reshape(NREM, CK)
    colrem = col[E2:].reshape(NREM, CK)
    r2 = lambda v: v.reshape(1, -1)
    ones = jnp.ones((CK,), jnp.float32)
    z_n = jnp.zeros((NP,), jnp.float32)
    z_rows = jnp.zeros((ZRA, HH), jnp.bfloat16)

    degp = _deg_sc(row2d, rowrem, ones, z_n).reshape(NC, NP)[0, :N].reshape(N, 1)
    q1, qb1 = _tcq(x, degp, r2(bnf_g), r2(bnf_b), Wf, r2(bn1_g), r2(bn1_b), W1)
    acc1 = _prop_sc(qb1, row2d, col2d, rowrem, colrem, z_rows)
    q2, qb2 = _tcmid(acc1, q1, degp, r2(b1), r2(bn2_g), r2(bn2_b), W2)
    acc2 = _prop_sc(qb2, row2d, col2d, rowrem, colrem, z_rows)
    q3, qb3 = _tcmid(acc2, q2, degp, r2(b2), r2(bn3_g), r2(bn3_b), W3)
    acc3 = _prop_sc(qb3, row2d, col2d, rowrem, colrem, z_rows)
    return _tcfin(acc3, q3, degp, r2(b3), batch.reshape(N, 1),
                  r2(bnfc_g), r2(bnfc_b), Wl, r2(bl), r2(bnh_g), r2(bnh_b),
                  Wc, r2(bc))


# speedup vs baseline: 1.4813x; 1.1003x over previous
import functools
import math

import jax
import jax.numpy as jnp
from jax import lax
from jax.experimental import pallas as pl
from jax.experimental.pallas import tpu as pltpu
from jax.experimental.pallas import tpu_sc as plsc

N = 10000
E = 320000
D = 128
H = 128
C = 10
G = 128
EPS = 1e-5
_RS = 1.0 / math.sqrt(1.0 + EPS)

NC = 2
NS = 16
HH = H // NC
NP = 10240
ZRA = 632
ZRB = N - (NS - 1) * ZRA
CK = 512
NCH = E // NS // CK
E2 = NS * NCH * CK
NREM = (E - E2) // CK


def _mesh():
    return plsc.VectorSubcoreMesh(
        core_axis_name="c", subcore_axis_name="s", num_cores=NC, num_subcores=NS
    )


@functools.partial(
    pl.kernel,
    out_type=jax.ShapeDtypeStruct((NC * NP,), jnp.float32),
    mesh=_mesh(),
    compiler_params=pltpu.CompilerParams(use_tc_tiling_on_sc=False),
    scratch_types=[
        pltpu.VMEM((NCH, CK), jnp.int32),
        pltpu.VMEM((NREM, CK), jnp.int32),
        pltpu.VMEM((CK,), jnp.float32),
        pltpu.VMEM_SHARED((NP,), jnp.float32),
    ],
)
def _deg_sc(r2_hbm, rr_hbm, ones_hbm, zeros_hbm, out_hbm, idx_v, idxr_v, ones_v, acc_sh):
    c = lax.axis_index("c")
    s = lax.axis_index("s")
    pltpu.sync_copy(ones_hbm, ones_v)

    @pl.when(s == 0)
    def _():
        pltpu.sync_copy(zeros_hbm, acc_sh)

    plsc.subcore_barrier()
    pltpu.sync_copy(r2_hbm.at[s], idx_v)

    def body(j, carry):
        pltpu.sync_copy(ones_v, acc_sh.at[idx_v.at[j]], add=True)
        return carry

    lax.fori_loop(0, NCH, body, 0)

    @pl.when(s == 0)
    def _():
        pltpu.sync_copy(rr_hbm, idxr_v)

        def rbody(j, carry):
            pltpu.sync_copy(ones_v, acc_sh.at[idxr_v.at[j]], add=True)
            return carry

        lax.fori_loop(0, NREM, rbody, 0)

    plsc.subcore_barrier()

    @pl.when(s == 0)
    def _():
        pltpu.sync_copy(acc_sh, out_hbm.at[pl.ds(c * NP, NP)])


@functools.partial(
    pl.kernel,
    out_type=jax.ShapeDtypeStruct((NC, N, HH), jnp.bfloat16),
    mesh=_mesh(),
    compiler_params=pltpu.CompilerParams(use_tc_tiling_on_sc=False),
    scratch_types=[
        pltpu.VMEM((NCH, CK), jnp.int32),
        pltpu.VMEM((NCH, CK), jnp.int32),
        pltpu.VMEM((NREM, CK), jnp.int32),
        pltpu.VMEM((NREM, CK), jnp.int32),
        pltpu.VMEM((CK, HH), jnp.bfloat16),
        pltpu.VMEM((CK, HH), jnp.bfloat16),
        pltpu.VMEM_SHARED((N, HH), jnp.bfloat16),
        pltpu.SemaphoreType.DMA,
        pltpu.SemaphoreType.DMA,
    ],
)
def _prop_sc(q_hbm, r2_hbm, c2_hbm, rr_hbm, cr_hbm, zrows_hbm, out_hbm,
             ir_v, ic_v, irr_v, icr_v, rows0, rows1, acc_sh, sem0, sem1):
    c = lax.axis_index("c")
    s = lax.axis_index("s")

    @pl.when(s < NS - 1)
    def _():
        pltpu.sync_copy(zrows_hbm, acc_sh.at[pl.ds(s * ZRA, ZRA)])

    @pl.when(s == NS - 1)
    def _():
        pltpu.sync_copy(
            zrows_hbm.at[pl.ds(0, ZRB)], acc_sh.at[pl.ds((NS - 1) * ZRA, ZRB)]
        )

    plsc.subcore_barrier()
    qh = q_hbm.at[c]
    pltpu.sync_copy(r2_hbm.at[s], ir_v)
    pltpu.sync_copy(c2_hbm.at[s], ic_v)
    pltpu.async_copy(qh.at[ir_v.at[0]], rows0, sem0)

    def body(t, carry):
        j0 = 2 * t
        pltpu.async_copy(qh.at[ir_v.at[j0 + 1]], rows1, sem1)
        pltpu.make_async_copy(qh.at[ir_v.at[j0]], rows0, sem0).wait()
        pltpu.sync_copy(rows0, acc_sh.at[ic_v.at[j0]], add=True)

        @pl.when(j0 + 2 < NCH)
        def _():
            pltpu.async_copy(qh.at[ir_v.at[j0 + 2]], rows0, sem0)

        pltpu.make_async_copy(qh.at[ir_v.at[j0 + 1]], rows1, sem1).wait()
        pltpu.sync_copy(rows1, acc_sh.at[ic_v.at[j0 + 1]], add=True)
        return carry

    lax.fori_loop(0, NCH // 2, body, 0)

    if NCH % 2 == 1:
        pltpu.make_async_copy(qh.at[ir_v.at[NCH - 1]], rows0, sem0).wait()
        pltpu.sync_copy(rows0, acc_sh.at[ic_v.at[NCH - 1]], add=True)

    @pl.when(s == 0)
    def _():
        pltpu.sync_copy(rr_hbm, irr_v)
        pltpu.sync_copy(cr_hbm, icr_v)

        def rbody(t, carry):
            pltpu.async_copy(qh.at[irr_v.at[t]], rows0, sem0).wait()
            pltpu.sync_copy(rows0, acc_sh.at[icr_v.at[t]], add=True)
            return carry

        lax.fori_loop(0, NREM, rbody, 0)

    plsc.subcore_barrier()

    @pl.when(s < NS - 1)
    def _():
        pltpu.sync_copy(
            acc_sh.at[pl.ds(s * ZRA, ZRA)], out_hbm.at[c].at[pl.ds(s * ZRA, ZRA)]
        )

    @pl.when(s == NS - 1)
    def _():
        pltpu.sync_copy(
            acc_sh.at[pl.ds((NS - 1) * ZRA, ZRB)],
            out_hbm.at[c].at[pl.ds((NS - 1) * ZRA, ZRB)],
        )


NB = 10
BN = N // NB

_bspec_h = pl.BlockSpec((BN, H), lambda i: (i, 0))
_bspec_d = pl.BlockSpec((BN, 1), lambda i: (i, 0))
_bspec_w = pl.BlockSpec((H, H), lambda i: (0, 0))
_bspec_v = pl.BlockSpec((1, H), lambda i: (0, 0))
_bspec_q = pl.BlockSpec((NC, BN, HH), lambda i: (0, i, 0))


def _tcq_body(x_ref, deg_ref, gf_ref, bf_ref, wf_ref, g_ref, b_ref, w_ref, q_ref, qb_ref):
    dis = lax.rsqrt(deg_ref[:] + 1.0)
    t0 = x_ref[:] * (_RS * gf_ref[:]) + bf_ref[:]
    h0 = jnp.maximum(jnp.dot(t0, wf_ref[:], preferred_element_type=jnp.float32), 0.0)
    t = h0 * (_RS * g_ref[:]) + b_ref[:]
    q = dis * jnp.dot(t, w_ref[:], preferred_element_type=jnp.float32)
    q_ref[0] = q[:, :HH]
    q_ref[1] = q[:, HH:]
    qb_ref[0] = q[:, :HH].astype(jnp.bfloat16)
    qb_ref[1] = q[:, HH:].astype(jnp.bfloat16)


_tcq = pl.pallas_call(
    _tcq_body,
    grid=(NB,),
    in_specs=[_bspec_h, _bspec_d, _bspec_v, _bspec_v, _bspec_w, _bspec_v,
              _bspec_v, _bspec_w],
    out_specs=(_bspec_q, _bspec_q),
    out_shape=(jax.ShapeDtypeStruct((NC, N, HH), jnp.float32),
               jax.ShapeDtypeStruct((NC, N, HH), jnp.bfloat16)),
)


def _tcmid_body(accp_ref, q_ref, deg_ref, bp_ref, g_ref, b_ref, w_ref,
                o_ref, ob_ref):
    dis = lax.rsqrt(deg_ref[:] + 1.0)
    m = jnp.concatenate(
        [accp_ref[0].astype(jnp.float32) + q_ref[0],
         accp_ref[1].astype(jnp.float32) + q_ref[1]], axis=1)
    h = jnp.maximum(dis * m + bp_ref[:], 0.0)
    t = h * (_RS * g_ref[:]) + b_ref[:]
    r = dis * jnp.dot(t, w_ref[:], preferred_element_type=jnp.float32)
    o_ref[0] = r[:, :HH]
    o_ref[1] = r[:, HH:]
    ob_ref[0] = r[:, :HH].astype(jnp.bfloat16)
    ob_ref[1] = r[:, HH:].astype(jnp.bfloat16)


_tcmid = pl.pallas_call(
    _tcmid_body,
    grid=(NB,),
    in_specs=[_bspec_q, _bspec_q, _bspec_d, _bspec_v, _bspec_v, _bspec_v,
              _bspec_w],
    out_specs=(_bspec_q, _bspec_q),
    out_shape=(jax.ShapeDtypeStruct((NC, N, HH), jnp.float32),
               jax.ShapeDtypeStruct((NC, N, HH), jnp.bfloat16)),
)


def _tcfin_body(accp_ref, q_ref, deg_ref, bp_ref, batch_ref, gfc_ref, bfc_ref,
                wl_ref, bl_ref, gh_ref, bh_ref, wc_ref, bc_ref, o_ref):
    dis = lax.rsqrt(deg_ref[:] + 1.0)
    m = jnp.concatenate(
        [accp_ref[0].astype(jnp.float32) + q_ref[0],
         accp_ref[1].astype(jnp.float32) + q_ref[1]], axis=1)
    h = jnp.maximum(dis * m + bp_ref[:], 0.0)
    ids = lax.broadcasted_iota(jnp.int32, (N, G), 1)
    onehot = jnp.where(batch_ref[:] == ids, 1.0, 0.0)
    dn = (((0,), (0,)), ((), ()))
    sums = lax.dot_general(onehot, h, dn, preferred_element_type=jnp.float32)
    cnt = lax.dot_general(
        onehot, jnp.ones((N, 1), jnp.float32), dn, preferred_element_type=jnp.float32
    )
    pooled = sums / jnp.maximum(cnt, 1.0)
    a = pooled * (_RS * gfc_ref[:]) + bfc_ref[:]
    a = jnp.maximum(
        jnp.dot(a, wl_ref[:], preferred_element_type=jnp.float32) + bl_ref[:], 0.0
    )
    a = a * (_RS * gh_ref[:]) + bh_ref[:]
    lg = jnp.dot(a, wc_ref[:], preferred_element_type=jnp.float32) + bc_ref[:]
    m = jnp.max(lg, axis=1, keepdims=True)
    o_ref[:] = (lg - m) - jnp.log(jnp.sum(jnp.exp(lg - m), axis=1, keepdims=True))


_tcfin = pl.pallas_call(_tcfin_body, out_shape=jax.ShapeDtypeStruct((G, C), jnp.float32))


def kernel(x, edge_index, batch, bnf_g, bnf_b, Wf, bn1_g, bn1_b, W1, b1,
           bn2_g, bn2_b, W2, b2, bn3_g, bn3_b, W3, b3, bnfc_g, bnfc_b, Wl, bl,
           bnh_g, bnh_b, Wc, bc):
    row = edge_index[0]
    col = edge_index[1]
    row2d = row[:E2].reshape(NS, NCH, CK)
    col2d = col[:E2].reshape(NS, NCH, CK)
    rowrem = row[E2:].reshape(NREM, CK)
    colrem = col[E2:].reshape(NREM, CK)
    r2 = lambda v: v.reshape(1, -1)
    ones = jnp.ones((CK,), jnp.float32)
    z_n = jnp.zeros((NP,), jnp.float32)
    z_rows = jnp.zeros((ZRA, HH), jnp.bfloat16)

    degp = _deg_sc(row2d, rowrem, ones, z_n).reshape(NC, NP)[0, :N].reshape(N, 1)
    q1, qb1 = _tcq(x, degp, r2(bnf_g), r2(bnf_b), Wf, r2(bn1_g), r2(bn1_b), W1)
    acc1 = _prop_sc(qb1, row2d, col2d, rowrem, colrem, z_rows)
    q2, qb2 = _tcmid(acc1, q1, degp, r2(b1), r2(bn2_g), r2(bn2_b), W2)
    acc2 = _prop_sc(qb2, row2d, col2d, rowrem, colrem, z_rows)
    q3, qb3 = _tcmid(acc2, q2, degp, r2(b2), r2(bn3_g), r2(bn3_b), W3)
    acc3 = _prop_sc(qb3, row2d, col2d, rowrem, colrem, z_rows)
    return _tcfin(acc3, q3, degp, r2(b3), batch.reshape(N, 1),
                  r2(bnfc_g), r2(bnfc_b), Wl, r2(bl), r2(bnh_g), r2(bnh_b),
                  Wc, r2(bc))


# speedup vs baseline: 1.5052x; 1.0161x over previous
import functools
import math

import jax
import jax.numpy as jnp
from jax import lax
from jax.experimental import pallas as pl
from jax.experimental.pallas import tpu as pltpu
from jax.experimental.pallas import tpu_sc as plsc

N = 10000
E = 320000
D = 128
H = 128
C = 10
G = 128
EPS = 1e-5
_RS = 1.0 / math.sqrt(1.0 + EPS)

NC = 2
NS = 16
HH = H // NC
NP = 10240
ZRA = 632
ZRB = N - (NS - 1) * ZRA
CK = 1000
NCH = E // NS // CK
E2 = NS * NCH * CK
NREM = (E - E2) // CK
NREMS = max(NREM, 1)


def _mesh():
    return plsc.VectorSubcoreMesh(
        core_axis_name="c", subcore_axis_name="s", num_cores=NC, num_subcores=NS
    )


@functools.partial(
    pl.kernel,
    out_type=jax.ShapeDtypeStruct((NC * NP,), jnp.float32),
    mesh=_mesh(),
    compiler_params=pltpu.CompilerParams(use_tc_tiling_on_sc=False),
    scratch_types=[
        pltpu.VMEM((NCH, CK), jnp.int32),
        pltpu.VMEM((NREMS, CK), jnp.int32),
        pltpu.VMEM((CK,), jnp.float32),
        pltpu.VMEM_SHARED((NP,), jnp.float32),
    ],
)
def _deg_sc(r2_hbm, rr_hbm, ones_hbm, zeros_hbm, out_hbm, idx_v, idxr_v, ones_v, acc_sh):
    c = lax.axis_index("c")
    s = lax.axis_index("s")
    pltpu.sync_copy(ones_hbm, ones_v)

    @pl.when(s == 0)
    def _():
        pltpu.sync_copy(zeros_hbm, acc_sh)

    plsc.subcore_barrier()
    pltpu.sync_copy(r2_hbm.at[s], idx_v)

    def body(j, carry):
        pltpu.sync_copy(ones_v, acc_sh.at[idx_v.at[j]], add=True)
        return carry

    lax.fori_loop(0, NCH, body, 0)

    if NREM:
        @pl.when(s == 0)
        def _():
            pltpu.sync_copy(rr_hbm, idxr_v)

            def rbody(j, carry):
                pltpu.sync_copy(ones_v, acc_sh.at[idxr_v.at[j]], add=True)
                return carry

            lax.fori_loop(0, NREM, rbody, 0)

    plsc.subcore_barrier()

    @pl.when(s == 0)
    def _():
        pltpu.sync_copy(acc_sh, out_hbm.at[pl.ds(c * NP, NP)])


@functools.partial(
    pl.kernel,
    out_type=jax.ShapeDtypeStruct((NC, N, HH), jnp.bfloat16),
    mesh=_mesh(),
    compiler_params=pltpu.CompilerParams(use_tc_tiling_on_sc=False),
    scratch_types=[
        pltpu.VMEM((NCH, CK), jnp.int32),
        pltpu.VMEM((NCH, CK), jnp.int32),
        pltpu.VMEM((NREMS, CK), jnp.int32),
        pltpu.VMEM((NREMS, CK), jnp.int32),
        pltpu.VMEM((CK, HH), jnp.bfloat16),
        pltpu.VMEM((CK, HH), jnp.bfloat16),
        pltpu.VMEM_SHARED((N, HH), jnp.bfloat16),
        pltpu.SemaphoreType.DMA,
        pltpu.SemaphoreType.DMA,
    ],
)
def _prop_sc(q_hbm, r2_hbm, c2_hbm, rr_hbm, cr_hbm, zrows_hbm, out_hbm,
             ir_v, ic_v, irr_v, icr_v, rows0, rows1, acc_sh, sem0, sem1):
    c = lax.axis_index("c")
    s = lax.axis_index("s")

    @pl.when(s < NS - 1)
    def _():
        pltpu.sync_copy(zrows_hbm, acc_sh.at[pl.ds(s * ZRA, ZRA)])

    @pl.when(s == NS - 1)
    def _():
        pltpu.sync_copy(
            zrows_hbm.at[pl.ds(0, ZRB)], acc_sh.at[pl.ds((NS - 1) * ZRA, ZRB)]
        )

    plsc.subcore_barrier()
    qh = q_hbm.at[c]
    pltpu.sync_copy(r2_hbm.at[s], ir_v)
    pltpu.sync_copy(c2_hbm.at[s], ic_v)
    pltpu.async_copy(qh.at[ir_v.at[0]], rows0, sem0)

    def body(t, carry):
        j0 = 2 * t
        pltpu.async_copy(qh.at[ir_v.at[j0 + 1]], rows1, sem1)
        pltpu.make_async_copy(qh.at[ir_v.at[j0]], rows0, sem0).wait()
        pltpu.sync_copy(rows0, acc_sh.at[ic_v.at[j0]], add=True)

        @pl.when(j0 + 2 < NCH)
        def _():
            pltpu.async_copy(qh.at[ir_v.at[j0 + 2]], rows0, sem0)

        pltpu.make_async_copy(qh.at[ir_v.at[j0 + 1]], rows1, sem1).wait()
        pltpu.sync_copy(rows1, acc_sh.at[ic_v.at[j0 + 1]], add=True)
        return carry

    lax.fori_loop(0, NCH // 2, body, 0)

    if NCH % 2 == 1:
        pltpu.make_async_copy(qh.at[ir_v.at[NCH - 1]], rows0, sem0).wait()
        pltpu.sync_copy(rows0, acc_sh.at[ic_v.at[NCH - 1]], add=True)

    if NREM:
        @pl.when(s == 0)
        def _():
            pltpu.sync_copy(rr_hbm, irr_v)
            pltpu.sync_copy(cr_hbm, icr_v)

            def rbody(t, carry):
                pltpu.async_copy(qh.at[irr_v.at[t]], rows0, sem0).wait()
                pltpu.sync_copy(rows0, acc_sh.at[icr_v.at[t]], add=True)
                return carry

            lax.fori_loop(0, NREM, rbody, 0)

    plsc.subcore_barrier()

    @pl.when(s < NS - 1)
    def _():
        pltpu.sync_copy(
            acc_sh.at[pl.ds(s * ZRA, ZRA)], out_hbm.at[c].at[pl.ds(s * ZRA, ZRA)]
        )

    @pl.when(s == NS - 1)
    def _():
        pltpu.sync_copy(
            acc_sh.at[pl.ds((NS - 1) * ZRA, ZRB)],
            out_hbm.at[c].at[pl.ds((NS - 1) * ZRA, ZRB)],
        )


NB = 10
BN = N // NB

_bspec_h = pl.BlockSpec((BN, H), lambda i: (i, 0))
_bspec_d = pl.BlockSpec((BN, 1), lambda i: (i, 0))
_bspec_w = pl.BlockSpec((H, H), lambda i: (0, 0))
_bspec_v = pl.BlockSpec((1, H), lambda i: (0, 0))
_bspec_q = pl.BlockSpec((NC, BN, HH), lambda i: (0, i, 0))


def _tcq_body(x_ref, deg_ref, gf_ref, bf_ref, wf_ref, g_ref, b_ref, w_ref, q_ref, qb_ref):
    dis = lax.rsqrt(deg_ref[:] + 1.0)
    t0 = x_ref[:] * (_RS * gf_ref[:]) + bf_ref[:]
    h0 = jnp.maximum(jnp.dot(t0, wf_ref[:], preferred_element_type=jnp.float32), 0.0)
    t = h0 * (_RS * g_ref[:]) + b_ref[:]
    q = dis * jnp.dot(t, w_ref[:], preferred_element_type=jnp.float32)
    q_ref[0] = q[:, :HH]
    q_ref[1] = q[:, HH:]
    qb_ref[0] = q[:, :HH].astype(jnp.bfloat16)
    qb_ref[1] = q[:, HH:].astype(jnp.bfloat16)


_tcq = pl.pallas_call(
    _tcq_body,
    grid=(NB,),
    in_specs=[_bspec_h, _bspec_d, _bspec_v, _bspec_v, _bspec_w, _bspec_v,
              _bspec_v, _bspec_w],
    out_specs=(_bspec_q, _bspec_q),
    out_shape=(jax.ShapeDtypeStruct((NC, N, HH), jnp.float32),
               jax.ShapeDtypeStruct((NC, N, HH), jnp.bfloat16)),
)


def _tcmid_body(accp_ref, q_ref, deg_ref, bp_ref, g_ref, b_ref, w_ref,
                o_ref, ob_ref):
    dis = lax.rsqrt(deg_ref[:] + 1.0)
    m = jnp.concatenate(
        [accp_ref[0].astype(jnp.float32) + q_ref[0],
         accp_ref[1].astype(jnp.float32) + q_ref[1]], axis=1)
    h = jnp.maximum(dis * m + bp_ref[:], 0.0)
    t = h * (_RS * g_ref[:]) + b_ref[:]
    r = dis * jnp.dot(t, w_ref[:], preferred_element_type=jnp.float32)
    o_ref[0] = r[:, :HH]
    o_ref[1] = r[:, HH:]
    ob_ref[0] = r[:, :HH].astype(jnp.bfloat16)
    ob_ref[1] = r[:, HH:].astype(jnp.bfloat16)


_tcmid = pl.pallas_call(
    _tcmid_body,
    grid=(NB,),
    in_specs=[_bspec_q, _bspec_q, _bspec_d, _bspec_v, _bspec_v, _bspec_v,
              _bspec_w],
    out_specs=(_bspec_q, _bspec_q),
    out_shape=(jax.ShapeDtypeStruct((NC, N, HH), jnp.float32),
               jax.ShapeDtypeStruct((NC, N, HH), jnp.bfloat16)),
)


def _tcfin_body(accp_ref, q_ref, deg_ref, bp_ref, batch_ref, gfc_ref, bfc_ref,
                wl_ref, bl_ref, gh_ref, bh_ref, wc_ref, bc_ref, o_ref):
    dis = lax.rsqrt(deg_ref[:] + 1.0)
    m = jnp.concatenate(
        [accp_ref[0].astype(jnp.float32) + q_ref[0],
         accp_ref[1].astype(jnp.float32) + q_ref[1]], axis=1)
    h = jnp.maximum(dis * m + bp_ref[:], 0.0)
    ids = lax.broadcasted_iota(jnp.int32, (N, G), 1)
    onehot = jnp.where(batch_ref[:] == ids, 1.0, 0.0)
    dn = (((0,), (0,)), ((), ()))
    sums = lax.dot_general(onehot, h, dn, preferred_element_type=jnp.float32)
    cnt = lax.dot_general(
        onehot, jnp.ones((N, 1), jnp.float32), dn, preferred_element_type=jnp.float32
    )
    pooled = sums / jnp.maximum(cnt, 1.0)
    a = pooled * (_RS * gfc_ref[:]) + bfc_ref[:]
    a = jnp.maximum(
        jnp.dot(a, wl_ref[:], preferred_element_type=jnp.float32) + bl_ref[:], 0.0
    )
    a = a * (_RS * gh_ref[:]) + bh_ref[:]
    lg = jnp.dot(a, wc_ref[:], preferred_element_type=jnp.float32) + bc_ref[:]
    m = jnp.max(lg, axis=1, keepdims=True)
    o_ref[:] = (lg - m) - jnp.log(jnp.sum(jnp.exp(lg - m), axis=1, keepdims=True))


_tcfin = pl.pallas_call(_tcfin_body, out_shape=jax.ShapeDtypeStruct((G, C), jnp.float32))


def kernel(x, edge_index, batch, bnf_g, bnf_b, Wf, bn1_g, bn1_b, W1, b1,
           bn2_g, bn2_b, W2, b2, bn3_g, bn3_b, W3, b3, bnfc_g, bnfc_b, Wl, bl,
           bnh_g, bnh_b, Wc, bc):
    row = edge_index[0]
    col = edge_index[1]
    row2d = row[:E2].reshape(NS, NCH, CK)
    col2d = col[:E2].reshape(NS, NCH, CK)
    if NREM:
        rowrem = row[E2:].reshape(NREM, CK)
        colrem = col[E2:].reshape(NREM, CK)
    else:
        rowrem = jnp.zeros((1, CK), jnp.int32)
        colrem = jnp.zeros((1, CK), jnp.int32)
    r2 = lambda v: v.reshape(1, -1)
    ones = jnp.ones((CK,), jnp.float32)
    z_n = jnp.zeros((NP,), jnp.float32)
    z_rows = jnp.zeros((ZRA, HH), jnp.bfloat16)

    degp = _deg_sc(row2d, rowrem, ones, z_n).reshape(NC, NP)[0, :N].reshape(N, 1)
    q1, qb1 = _tcq(x, degp, r2(bnf_g), r2(bnf_b), Wf, r2(bn1_g), r2(bn1_b), W1)
    acc1 = _prop_sc(qb1, row2d, col2d, rowrem, colrem, z_rows)
    q2, qb2 = _tcmid(acc1, q1, degp, r2(b1), r2(bn2_g), r2(bn2_b), W2)
    acc2 = _prop_sc(qb2, row2d, col2d, rowrem, colrem, z_rows)
    q3, qb3 = _tcmid(acc2, q2, degp, r2(b2), r2(bn3_g), r2(bn3_b), W3)
    acc3 = _prop_sc(qb3, row2d, col2d, rowrem, colrem, z_rows)
    return _tcfin(acc3, q3, degp, r2(b3), batch.reshape(N, 1),
                  r2(bnfc_g), r2(bnfc_b), Wl, r2(bl), r2(bnh_g), r2(bnh_b),
                  Wc, r2(bc))
